# baseline (speedup 1.0000x reference)
import functools

import jax
import jax.numpy as jnp
from jax import lax
from jax.experimental import pallas as pl
from jax.experimental.pallas import tpu as pltpu
from jax.experimental.pallas import tpu_sc as plsc

B, H, W, C = 256, 16, 16, 256
NUM_CLASSES = 10
FLAT = H * W * C
CH = 1024
NCH = FLAT // CH
NC, NS = 2, 16
NW = NC * NS
RW = B // NW
GROUP = 16
NG = RW * NCH // GROUP
GPR = NCH // GROUP
SLICES = GROUP * CH // 16


def _body(t_hbm, lab_hbm, tab_hbm, out_hbm, lab_v, t_buf, e_buf,
          sem_t, sem_e, sem_o):
    wid = lax.axis_index("s") * NC + lax.axis_index("c")
    base2 = wid * (RW * NCH)
    base_row = wid * RW

    pltpu.sync_copy(lab_hbm, lab_v)

    def tensor_copy(g, p):
        return pltpu.make_async_copy(
            t_hbm.at[pl.ds(base2 + g * GROUP, GROUP)], t_buf.at[p],
            sem_t.at[p])

    def gather_copy(g, p):
        lab = plsc.load_gather(
            lab_v, [jnp.full((16,), base_row + g // GPR, jnp.int32)])
        gidx = lab * NCH + (g % GPR) * GROUP + lax.iota(jnp.int32, 16)
        return pltpu.make_async_copy(tab_hbm.at[gidx], e_buf.at[p],
                                     sem_e.at[p])

    def store_copy(g, p):
        return pltpu.make_async_copy(
            t_buf.at[p], out_hbm.at[pl.ds(base2 + g * GROUP, GROUP)],
            sem_o.at[p])

    tensor_copy(0, 0).start()
    gather_copy(0, 0).start()

    def group_body(g, carry):
        p = g % 2
        q = 1 - p

        @pl.when(g >= 1)
        def _():
            store_copy(g - 1, q).wait()

        @pl.when(g + 1 < NG)
        def _():
            tensor_copy(g + 1, q).start()
            gather_copy(g + 1, q).start()

        tensor_copy(g, p).wait()
        gather_copy(g, p).wait()

        def add_body(k, carry2):
            j = k // (CH // 16)
            m = k % (CH // 16)
            sl = pl.ds(m * 16, 16)
            t_buf[p, j, sl] = t_buf[p, j, sl] + e_buf[p, j, sl]
            return carry2

        lax.fori_loop(0, SLICES, add_body, None, unroll=8)

        store_copy(g, p).start()
        return carry

    lax.fori_loop(0, NG, group_body, None)
    store_copy(NG - 1, (NG - 1) % 2).wait()


@jax.jit
def _run(t2, labels, tab2):
    kfn = pl.kernel(
        _body,
        out_type=jax.ShapeDtypeStruct((B * NCH, CH), jnp.float32),
        mesh=plsc.VectorSubcoreMesh(core_axis_name="c", subcore_axis_name="s",
                                    num_cores=NC, num_subcores=NS),
        scratch_types=[
            pltpu.VMEM((B,), jnp.int32),
            pltpu.VMEM((2, GROUP, CH), jnp.float32),
            pltpu.VMEM((2, GROUP, CH), jnp.float32),
            pltpu.SemaphoreType.DMA((2,)),
            pltpu.SemaphoreType.DMA((2,)),
            pltpu.SemaphoreType.DMA((2,)),
        ],
        compiler_params=pltpu.CompilerParams(needs_layout_passes=False),
    )
    return kfn(t2, labels, tab2)


def kernel(tensor, labels, embed_table):
    t2 = tensor.reshape(B * NCH, CH)
    tab2 = embed_table.reshape(NUM_CLASSES * NCH, CH)
    out2 = _run(t2, labels.astype(jnp.int32), tab2)
    return out2.reshape(B, H, W, C)


# speedup vs baseline: 1.1836x; 1.1836x over previous
import jax
import jax.numpy as jnp
from jax import lax
from jax.experimental import pallas as pl
from jax.experimental.pallas import tpu as pltpu
from jax.experimental.pallas import tpu_sc as plsc

B, H, W, C = 256, 16, 16, 256
NUM_CLASSES = 10
FLAT = H * W * C
CH = 1024
NCH = FLAT // CH
NC, NS = 2, 16
NW = NC * NS
RW = B // NW
SLABS = 4
GROUP = 16
GPR = H // SLABS
NG = RW * GPR
SLICES = GROUP * CH // 16


def _body(t_hbm, lab_hbm, tab_hbm, out_hbm, lab_v, t_buf, e_buf,
          sem_t, sem_e, sem_o):
    wid = lax.axis_index("s") * NC + lax.axis_index("c")
    base_row = wid * RW

    pltpu.sync_copy(lab_hbm, lab_v)

    def tensor_copy(g, p):
        return pltpu.make_async_copy(
            t_hbm.at[base_row + g // GPR, pl.ds((g % GPR) * SLABS, SLABS)],
            t_buf.at[p], sem_t.at[p])

    def gather_copy(g, p):
        lab = plsc.load_gather(
            lab_v, [jnp.full((16,), base_row + g // GPR, jnp.int32)])
        gidx = lab * NCH + (g % GPR) * GROUP + lax.iota(jnp.int32, 16)
        return pltpu.make_async_copy(tab_hbm.at[gidx], e_buf.at[p],
                                     sem_e.at[p])

    def store_copy(g, p):
        return pltpu.make_async_copy(
            t_buf.at[p],
            out_hbm.at[base_row + g // GPR, pl.ds((g % GPR) * SLABS, SLABS)],
            sem_o.at[p])

    tensor_copy(0, 0).start()
    gather_copy(0, 0).start()

    def group_body(g, carry):
        p = g % 2
        q = 1 - p

        @pl.when(g >= 1)
        def _():
            store_copy(g - 1, q).wait()

        @pl.when(g + 1 < NG)
        def _():
            tensor_copy(g + 1, q).start()
            gather_copy(g + 1, q).start()

        tensor_copy(g, p).wait()
        gather_copy(g, p).wait()

        def add_body(k, carry2):
            i = k // (W * C // 16)
            j = (k // (C // 16)) % W
            m = k % (C // 16)
            ck = k // (CH // 16)
            cm = k % (CH // 16)
            sl = pl.ds(m * 16, 16)
            t_buf[p, i, j, sl] = (t_buf[p, i, j, sl]
                                  + e_buf[p, ck, pl.ds(cm * 16, 16)])
            return carry2

        lax.fori_loop(0, SLICES, add_body, None, unroll=8)

        store_copy(g, p).start()
        return carry

    lax.fori_loop(0, NG, group_body, None)
    store_copy(NG - 1, (NG - 1) % 2).wait()


@jax.jit
def _run(tensor, labels, tab2):
    kfn = pl.kernel(
        _body,
        out_type=jax.ShapeDtypeStruct((B, H, W, C), jnp.float32),
        mesh=plsc.VectorSubcoreMesh(core_axis_name="c", subcore_axis_name="s",
                                    num_cores=NC, num_subcores=NS),
        scratch_types=[
            pltpu.VMEM((B,), jnp.int32),
            pltpu.VMEM((2, SLABS, W, C), jnp.float32),
            pltpu.VMEM((2, GROUP, CH), jnp.float32),
            pltpu.SemaphoreType.DMA((2,)),
            pltpu.SemaphoreType.DMA((2,)),
            pltpu.SemaphoreType.DMA((2,)),
        ],
        compiler_params=pltpu.CompilerParams(needs_layout_passes=False),
    )
    return kfn(tensor, labels, tab2)


def kernel(tensor, labels, embed_table):
    tab2 = embed_table.reshape(NUM_CLASSES * NCH, CH)
    return _run(tensor, labels.astype(jnp.int32), tab2)


# speedup vs baseline: 1.4277x; 1.2062x over previous
import jax
import jax.numpy as jnp
from jax import lax
from jax.experimental import pallas as pl
from jax.experimental.pallas import tpu as pltpu
from jax.experimental.pallas import tpu_sc as plsc

B, H, W, C = 256, 16, 16, 256
NUM_CLASSES = 10
FLAT = H * W * C
CH = 1024
NCH = FLAT // CH
NC, NS = 2, 16
NW = NC * NS
RW = B // NW
SLABS = 4
GROUP = 16
GPR = H // SLABS
NG = RW * GPR
SLICES = GROUP * CH // 16


def _body(t_hbm, lab_hbm, tab_hbm, out_hbm, lab_v, t_buf, e_buf,
          sem_t, sem_e, sem_o):
    wid = lax.axis_index("s") * NC + lax.axis_index("c")
    base_row = wid * RW

    pltpu.sync_copy(lab_hbm, lab_v)

    def tensor_copy(g, p):
        return pltpu.make_async_copy(
            t_hbm.at[base_row + g // GPR, pl.ds((g % GPR) * SLABS, SLABS)],
            t_buf.at[p], sem_t.at[p])

    def gather_copy(g, p):
        lab = plsc.load_gather(
            lab_v, [jnp.full((16,), base_row + g // GPR, jnp.int32)])
        gidx = lab * NCH + (g % GPR) * GROUP + lax.iota(jnp.int32, 16)
        return pltpu.make_async_copy(tab_hbm.at[gidx], e_buf.at[p],
                                     sem_e.at[p])

    def store_copy(g, p):
        return pltpu.make_async_copy(
            t_buf.at[p],
            out_hbm.at[base_row + g // GPR, pl.ds((g % GPR) * SLABS, SLABS)],
            sem_o.at[p])

    tensor_copy(0, 0).start()
    gather_copy(0, 0).start()

    def group_body(g, carry):
        p = g % 2
        q = 1 - p

        @pl.when(g >= 1)
        def _():
            store_copy(g - 1, q).wait()

        @pl.when(g + 1 < NG)
        def _():
            tensor_copy(g + 1, q).start()
            gather_copy(g + 1, q).start()

        tensor_copy(g, p).wait()
        gather_copy(g, p).wait()

        for i in range(SLABS):
            def j_body(j, carry2, i=i):
                ck = i * SLABS + j // SLABS
                cw = j % SLABS
                for m in range(C // 16):
                    sl = pl.ds(m * 16, 16)
                    plsc.addupdate(t_buf.at[p, i, j, sl],
                                   e_buf[p, ck, cw, sl])
                return carry2

            lax.fori_loop(0, W, j_body, None)

        store_copy(g, p).start()
        return carry

    lax.fori_loop(0, NG, group_body, None)
    store_copy(NG - 1, (NG - 1) % 2).wait()


@jax.jit
def _run(tensor, labels, tab2):
    kfn = pl.kernel(
        _body,
        out_type=jax.ShapeDtypeStruct((B, H, W, C), jnp.float32),
        mesh=plsc.VectorSubcoreMesh(core_axis_name="c", subcore_axis_name="s",
                                    num_cores=NC, num_subcores=NS),
        scratch_types=[
            pltpu.VMEM((B,), jnp.int32),
            pltpu.VMEM((2, SLABS, W, C), jnp.float32),
            pltpu.VMEM((2, GROUP, SLABS, C), jnp.float32),
            pltpu.SemaphoreType.DMA((2,)),
            pltpu.SemaphoreType.DMA((2,)),
            pltpu.SemaphoreType.DMA((2,)),
        ],
        compiler_params=pltpu.CompilerParams(needs_layout_passes=False),
    )
    return kfn(tensor, labels, tab2)


def kernel(tensor, labels, embed_table):
    tab2 = embed_table.reshape(NUM_CLASSES * NCH, SLABS, C)
    return _run(tensor, labels.astype(jnp.int32), tab2)


# speedup vs baseline: 2.4343x; 1.7050x over previous
import jax
import jax.numpy as jnp
from jax import lax
from jax.experimental import pallas as pl
from jax.experimental.pallas import tpu as pltpu
from jax.experimental.pallas import tpu_sc as plsc

B, H, W, C = 256, 16, 16, 256
NUM_CLASSES = 10
FLAT = H * W * C
CH = 1024
NCH = FLAT // CH
NC, NS = 2, 16
NW = NC * NS
RW = B // NW
SLABS = 4
GROUP = 16
GPR = H // SLABS
NG = RW * GPR
SLICES = GROUP * CH // 16


def _body(t_hbm, lab_hbm, tab_hbm, out_hbm, lab_v, t_buf, e_buf,
          sem_t, sem_e, sem_o):
    wid = lax.axis_index("s") * NC + lax.axis_index("c")
    base_row = wid * RW

    pltpu.sync_copy(lab_hbm, lab_v)

    def tensor_copy(g, p):
        return pltpu.make_async_copy(
            t_hbm.at[base_row + g // GPR, pl.ds((g % GPR) * SLABS, SLABS)],
            t_buf.at[p], sem_t.at[p])

    def gather_copy(g, p):
        lab = plsc.load_gather(
            lab_v, [jnp.full((16,), base_row + g // GPR, jnp.int32)])
        gidx = lab * NCH + (g % GPR) * GROUP + lax.iota(jnp.int32, 16)
        return pltpu.make_async_copy(tab_hbm.at[gidx], e_buf.at[p],
                                     sem_e.at[p])

    def store_copy(g, p):
        return pltpu.make_async_copy(
            t_buf.at[p],
            out_hbm.at[base_row + g // GPR, pl.ds((g % GPR) * SLABS, SLABS)],
            sem_o.at[p])

    tensor_copy(0, 0).start()
    gather_copy(0, 0).start()

    def group_body(g, carry):
        p = g % 2
        q = 1 - p

        @pl.when(g >= 1)
        def _():
            store_copy(g - 1, q).wait()

        @pl.when(g + 1 < NG)
        def _():
            tensor_copy(g + 1, q).start()
            gather_copy(g + 1, q).start()

        tensor_copy(g, p).wait()
        gather_copy(g, p).wait()

        for i in range(SLABS):
            def j_body(j, carry2, i=i):
                ck = i * SLABS + j // SLABS
                cw = j % SLABS
                vals = [e_buf[p, ck, cw, pl.ds(m * 16, 16)]
                        for m in range(C // 16)]
                for m in range(C // 16):
                    plsc.addupdate(t_buf.at[p, i, j, pl.ds(m * 16, 16)],
                                   vals[m])
                return carry2

            lax.fori_loop(0, W, j_body, None)

        store_copy(g, p).start()
        return carry

    lax.fori_loop(0, NG, group_body, None)
    store_copy(NG - 1, (NG - 1) % 2).wait()


@jax.jit
def _run(tensor, labels, tab2):
    kfn = pl.kernel(
        _body,
        out_type=jax.ShapeDtypeStruct((B, H, W, C), jnp.float32),
        mesh=plsc.VectorSubcoreMesh(core_axis_name="c", subcore_axis_name="s",
                                    num_cores=NC, num_subcores=NS),
        scratch_types=[
            pltpu.VMEM((B,), jnp.int32),
            pltpu.VMEM((2, SLABS, W, C), jnp.float32),
            pltpu.VMEM((2, GROUP, SLABS, C), jnp.float32),
            pltpu.SemaphoreType.DMA((2,)),
            pltpu.SemaphoreType.DMA((2,)),
            pltpu.SemaphoreType.DMA((2,)),
        ],
        compiler_params=pltpu.CompilerParams(needs_layout_passes=False),
    )
    return kfn(tensor, labels, tab2)


def kernel(tensor, labels, embed_table):
    tab2 = embed_table.reshape(NUM_CLASSES * NCH, SLABS, C)
    return _run(tensor, labels.astype(jnp.int32), tab2)


# speedup vs baseline: 2.7154x; 1.1155x over previous
import jax
import jax.numpy as jnp
from jax import lax
from jax.experimental import pallas as pl
from jax.experimental.pallas import tpu as pltpu
from jax.experimental.pallas import tpu_sc as plsc

B, H, W, C = 256, 16, 16, 256
NUM_CLASSES = 10
FLAT = H * W * C
CH = 1024
NCH = FLAT // CH
NC, NS = 2, 16
NW = NC * NS
RW = B // NW
SLABS = 4
GROUP = 16
GPR = H // SLABS
NG = RW * GPR
SLICES = GROUP * CH // 16


def _body(t_hbm, lab_hbm, tab_hbm, out_hbm, lab_v, t_buf, e_buf, tab_s,
          sem_t, sem_e, sem_o):
    sid = lax.axis_index("s")
    wid = sid * NC + lax.axis_index("c")
    base_row = wid * RW

    stripe = NUM_CLASSES * NCH // NS
    pltpu.sync_copy(tab_hbm.at[pl.ds(sid * stripe, stripe)],
                    tab_s.at[pl.ds(sid * stripe, stripe)])
    pltpu.sync_copy(lab_hbm, lab_v.at[pl.ds(0, B)])
    plsc.subcore_barrier()

    def tensor_copy(g, p):
        return pltpu.make_async_copy(
            t_hbm.at[base_row + g // GPR, pl.ds((g % GPR) * SLABS, SLABS)],
            t_buf.at[p], sem_t.at[p])

    def gather_copy(g, p):
        lab = lab_v[pl.ds(base_row + g // GPR, 16)][0]
        return pltpu.make_async_copy(
            tab_s.at[pl.ds(lab * NCH + (g % GPR) * GROUP, GROUP)],
            e_buf.at[p], sem_e.at[p])

    def store_copy(g, p):
        return pltpu.make_async_copy(
            t_buf.at[p],
            out_hbm.at[base_row + g // GPR, pl.ds((g % GPR) * SLABS, SLABS)],
            sem_o.at[p])

    tensor_copy(0, 0).start()
    gather_copy(0, 0).start()

    def group_body(g, carry):
        p = g % 2
        q = 1 - p

        @pl.when(g >= 1)
        def _():
            store_copy(g - 1, q).wait()

        @pl.when(g + 1 < NG)
        def _():
            tensor_copy(g + 1, q).start()
            gather_copy(g + 1, q).start()

        tensor_copy(g, p).wait()
        gather_copy(g, p).wait()

        for i in range(SLABS):
            def j_body(j, carry2, i=i):
                ck = i * SLABS + j // SLABS
                cw = j % SLABS
                vals = [e_buf[p, ck, cw, pl.ds(m * 16, 16)]
                        for m in range(C // 16)]
                for m in range(C // 16):
                    plsc.addupdate(t_buf.at[p, i, j, pl.ds(m * 16, 16)],
                                   vals[m])
                return carry2

            lax.fori_loop(0, W, j_body, None)

        store_copy(g, p).start()
        return carry

    lax.fori_loop(0, NG, group_body, None)
    store_copy(NG - 1, (NG - 1) % 2).wait()


@jax.jit
def _run(tensor, labels, tab2):
    kfn = pl.kernel(
        _body,
        out_type=jax.ShapeDtypeStruct((B, H, W, C), jnp.float32),
        mesh=plsc.VectorSubcoreMesh(core_axis_name="c", subcore_axis_name="s",
                                    num_cores=NC, num_subcores=NS),
        scratch_types=[
            pltpu.VMEM((B + 16,), jnp.int32),
            pltpu.VMEM((2, SLABS, W, C), jnp.float32),
            pltpu.VMEM((2, GROUP, SLABS, C), jnp.float32),
            pltpu.VMEM_SHARED((NUM_CLASSES * NCH, SLABS, C), jnp.float32),
            pltpu.SemaphoreType.DMA((2,)),
            pltpu.SemaphoreType.DMA((2,)),
            pltpu.SemaphoreType.DMA((2,)),
        ],
        compiler_params=pltpu.CompilerParams(needs_layout_passes=False),
    )
    return kfn(tensor, labels, tab2)


def kernel(tensor, labels, embed_table):
    tab2 = embed_table.reshape(NUM_CLASSES * NCH, SLABS, C)
    return _run(tensor, labels.astype(jnp.int32), tab2)


# speedup vs baseline: 3.0096x; 1.1083x over previous
import jax
import jax.numpy as jnp
from jax import lax
from jax.experimental import pallas as pl
from jax.experimental.pallas import tpu as pltpu
from jax.experimental.pallas import tpu_sc as plsc

B, H, W, C = 256, 16, 16, 256
NUM_CLASSES = 10
FLAT = H * W * C
SLABS = 2
CH = SLABS * C
NCH = FLAT // CH
NC, NS = 2, 16
NW = NC * NS
RW = B // NW
GROUP = SLABS * W * C // CH
GPR = H // SLABS
NG = RW * GPR
NBUF = 4
PF = NBUF - 1


def _body(t_hbm, lab_hbm, tab_hbm, out_hbm, lab_v, t_buf, e_buf, tab_s,
          sem_t, sem_e, sem_o):
    sid = lax.axis_index("s")
    wid = sid * NC + lax.axis_index("c")
    base_row = wid * RW

    stripe = NUM_CLASSES * NCH // NS
    pltpu.sync_copy(tab_hbm.at[pl.ds(sid * stripe, stripe)],
                    tab_s.at[pl.ds(sid * stripe, stripe)])
    pltpu.sync_copy(lab_hbm, lab_v.at[pl.ds(0, B)])
    plsc.subcore_barrier()

    def tensor_copy(g, p):
        return pltpu.make_async_copy(
            t_hbm.at[base_row + g // GPR, pl.ds((g % GPR) * SLABS, SLABS)],
            t_buf.at[p], sem_t.at[p])

    def gather_copy(g, p):
        lab = lab_v[pl.ds(base_row + g // GPR, 16)][0]
        return pltpu.make_async_copy(
            tab_s.at[pl.ds(lab * NCH + (g % GPR) * GROUP, GROUP)],
            e_buf.at[p], sem_e.at[p])

    def store_copy(g, p):
        return pltpu.make_async_copy(
            t_buf.at[p],
            out_hbm.at[base_row + g // GPR, pl.ds((g % GPR) * SLABS, SLABS)],
            sem_o.at[p])

    for k in range(PF):
        tensor_copy(k, k).start()
        gather_copy(k, k).start()

    def group_body(g, carry):
        p = g % NBUF

        tensor_copy(g, p).wait()
        gather_copy(g, p).wait()

        NM = C // 16

        for i in range(SLABS):
            @plsc.parallel_loop(0, W, 1, unroll=2)
            def _j(j, i=i):
                ck = i * (W // SLABS) + j // SLABS
                cw = j % SLABS
                vals = [e_buf[p, ck, cw, pl.ds(m * 16, 16)]
                        for m in range(NM)]
                for m in range(NM):
                    plsc.addupdate(t_buf.at[p, i, j, pl.ds(m * 16, 16)],
                                   vals[m])

        store_copy(g, p).start()

        @pl.when(jnp.logical_and(g >= 1, g + PF < NG))
        def _():
            store_copy(g - 1, (g - 1) % NBUF).wait()

        @pl.when(g + PF < NG)
        def _():
            tensor_copy(g + PF, (g + PF) % NBUF).start()
            gather_copy(g + PF, (g + PF) % NBUF).start()

        return carry

    lax.fori_loop(0, NG, group_body, None)
    for k in range(NG - NBUF, NG):
        store_copy(k, k % NBUF).wait()


@jax.jit
def _run(tensor, labels, tab2):
    kfn = pl.kernel(
        _body,
        out_type=jax.ShapeDtypeStruct((B, H, W, C), jnp.float32),
        mesh=plsc.VectorSubcoreMesh(core_axis_name="c", subcore_axis_name="s",
                                    num_cores=NC, num_subcores=NS),
        scratch_types=[
            pltpu.VMEM((B + 16,), jnp.int32),
            pltpu.VMEM((NBUF, SLABS, W, C), jnp.float32),
            pltpu.VMEM((NBUF, GROUP, SLABS, C), jnp.float32),
            pltpu.VMEM_SHARED((NUM_CLASSES * NCH, SLABS, C), jnp.float32),
            pltpu.SemaphoreType.DMA((NBUF,)),
            pltpu.SemaphoreType.DMA((NBUF,)),
            pltpu.SemaphoreType.DMA((NBUF,)),
        ],
        compiler_params=pltpu.CompilerParams(needs_layout_passes=False),
    )
    return kfn(tensor, labels, tab2)


def kernel(tensor, labels, embed_table):
    tab2 = embed_table.reshape(NUM_CLASSES * NCH, SLABS, C)
    return _run(tensor, labels.astype(jnp.int32), tab2)
